# Initial kernel scaffold; baseline (speedup 1.0000x reference)
import functools

import jax
import jax.numpy as jnp
from jax import lax
from jax.experimental import pallas as pl
from jax.experimental.pallas import tpu as pltpu
from jax.experimental.pallas import tpu_sc as plsc

_DIM = 64
_B = 16384
_L = 200
_NC = 2
_NS = 16
_NW = _NC * _NS
_RPW = _B // _NW
_HALF = _RPW // 2
_CHUNK = _L // 2


def _make_kernel():
  mesh = plsc.VectorSubcoreMesh(core_axis_name="c", subcore_axis_name="s")

  @functools.partial(
      pl.kernel,
      mesh=mesh,
      out_type=jax.ShapeDtypeStruct((_B, _DIM), jnp.float32),
      scratch_types=[
          pltpu.VMEM((2 * _HALF, _CHUNK), jnp.int32),
          pltpu.VMEM((_L, _DIM), jnp.float32),
          pltpu.VMEM((_L, _DIM), jnp.float32),
          pltpu.VMEM((_HALF, _DIM), jnp.float32),
          pltpu.SemaphoreType.DMA,
          pltpu.SemaphoreType.DMA,
      ],
  )
  def k(x_hbm, table_hbm, out_hbm, idx_v, rows0, rows1, out_v, sem0, sem1):
    wid = lax.axis_index("s") * _NC + lax.axis_index("c")
    base = wid * _RPW
    bufs = (rows0, rows1)
    sems = (sem0, sem1)

    def fire(b, buf, sem):
      pltpu.make_async_copy(
          table_hbm.at[idx_v.at[2 * b]],
          buf.at[pl.ds(0, _CHUNK)], sem).start()
      pltpu.make_async_copy(
          table_hbm.at[idx_v.at[2 * b + 1]],
          buf.at[pl.ds(_CHUNK, _CHUNK)], sem).start()

    def drain(b, buf, sem):
      pltpu.make_async_copy(
          table_hbm.at[idx_v.at[2 * b]],
          buf.at[pl.ds(0, _CHUNK)], sem).wait()
      pltpu.make_async_copy(
          table_hbm.at[idx_v.at[2 * b + 1]],
          buf.at[pl.ds(_CHUNK, _CHUNK)], sem).wait()

    for h in range(2):
      row0 = base + h * _HALF
      pltpu.sync_copy(x_hbm.at[pl.ds(2 * row0, 2 * _HALF), :], idx_v)
      fire(0, bufs[0], sems[0])

      def outer(i, _):
        for kk in range(2):
          b = 2 * i + kk
          buf = bufs[kk]
          nxt = b + 1

          @pl.when(nxt < _HALF)
          def _():
            fire(nxt, bufs[1 - kk], sems[1 - kk])

          drain(b, buf, sems[kk])

          def jbody(j, accs):
            return tuple(
                accs[d] + buf[j, pl.ds(16 * d, 16)] for d in range(4))

          z = jnp.zeros((16,), jnp.float32)
          accs = lax.fori_loop(0, _L, jbody, (z, z, z, z))
          scale = jnp.float32(1.0 / _L)
          for d in range(4):
            out_v[b, pl.ds(16 * d, 16)] = accs[d] * scale
        return 0

      lax.fori_loop(0, _HALF // 2, outer, 0)
      pltpu.sync_copy(out_v, out_hbm.at[pl.ds(row0, _HALF), :])

  return k


_sc_kernel = _make_kernel()


def kernel(x, table):
  x2 = x.astype(jnp.int32).reshape(2 * _B, _CHUNK)
  return _sc_kernel(x2, table)


# speedup vs baseline: 2.7948x; 2.7948x over previous
import functools

import jax
import jax.numpy as jnp
from jax import lax
from jax.experimental import pallas as pl
from jax.experimental.pallas import tpu as pltpu
from jax.experimental.pallas import tpu_sc as plsc

_DIM = 64
_B = 16384
_L = 200
_NC = 2
_NS = 16
_NW = _NC * _NS
_RPW = _B // _NW
_HALF = _RPW // 2
_CHUNK = _L // 2


def _make_kernel():
  mesh = plsc.VectorSubcoreMesh(core_axis_name="c", subcore_axis_name="s")

  @functools.partial(
      pl.kernel,
      mesh=mesh,
      compiler_params=pltpu.CompilerParams(use_tc_tiling_on_sc=False),
      out_type=jax.ShapeDtypeStruct((_B, _DIM), jnp.float32),
      scratch_types=[
          pltpu.VMEM((2 * _HALF, _CHUNK), jnp.int32),
          pltpu.VMEM((_L, _DIM), jnp.float32),
          pltpu.VMEM((_L, _DIM), jnp.float32),
          pltpu.VMEM((_HALF, _DIM), jnp.float32),
          pltpu.SemaphoreType.DMA,
          pltpu.SemaphoreType.DMA,
      ],
  )
  def k(x_hbm, table_hbm, out_hbm, idx_v, rows0, rows1, out_v, sem0, sem1):
    wid = lax.axis_index("s") * _NC + lax.axis_index("c")
    base = wid * _RPW
    bufs = (rows0, rows1)
    sems = (sem0, sem1)

    def fire(b, buf, sem):
      pltpu.make_async_copy(
          table_hbm.at[idx_v.at[2 * b]],
          buf.at[pl.ds(0, _CHUNK)], sem).start()
      pltpu.make_async_copy(
          table_hbm.at[idx_v.at[2 * b + 1]],
          buf.at[pl.ds(_CHUNK, _CHUNK)], sem).start()

    def drain(b, buf, sem):
      pltpu.make_async_copy(
          table_hbm.at[idx_v.at[2 * b]],
          buf.at[pl.ds(0, _CHUNK)], sem).wait()
      pltpu.make_async_copy(
          table_hbm.at[idx_v.at[2 * b + 1]],
          buf.at[pl.ds(_CHUNK, _CHUNK)], sem).wait()

    for h in range(2):
      row0 = base + h * _HALF
      pltpu.sync_copy(x_hbm.at[pl.ds(2 * row0, 2 * _HALF), :], idx_v)
      fire(0, bufs[0], sems[0])

      def outer(i, _):
        for kk in range(2):
          b = 2 * i + kk
          buf = bufs[kk]
          nxt = b + 1

          @pl.when(nxt < _HALF)
          def _():
            fire(nxt, bufs[1 - kk], sems[1 - kk])

          drain(b, buf, sems[kk])

          def jbody(j, accs):
            return tuple(
                accs[d] + buf[j, pl.ds(16 * d, 16)] for d in range(4))

          z = jnp.zeros((16,), jnp.float32)
          accs = lax.fori_loop(0, _L, jbody, (z, z, z, z))
          scale = jnp.float32(1.0 / _L)
          for d in range(4):
            out_v[b, pl.ds(16 * d, 16)] = accs[d] * scale
        return 0

      lax.fori_loop(0, _HALF // 2, outer, 0)
      pltpu.sync_copy(out_v, out_hbm.at[pl.ds(row0, _HALF), :])

  return k


_sc_kernel = _make_kernel()


def kernel(x, table):
  x2 = x.astype(jnp.int32).reshape(2 * _B, _CHUNK)
  return _sc_kernel(x2, table)
